# baseline (speedup 1.0000x reference)
import jax
import jax.numpy as jnp
from jax import lax
from jax.experimental import pallas as pl
from jax.experimental.pallas import tpu as pltpu
from jax.experimental.pallas import tpu_sc as plsc

VOCAB = 1000000
EMBED = 64
B = 4096
L = 200

NUM_IDX = B * L
NC, NS = 2, 16
NW = NC * NS
PER_W = NUM_IDX // NW
WIN = 128
NWIN = PER_W // WIN
NBUF = 2
PAIR_LANES = 2 * EMBED
VREG = 16


def _sc_gather(table_pairs, idx_half, half_off):
    mesh = plsc.VectorSubcoreMesh(core_axis_name="c", subcore_axis_name="s")

    scratch = [
        pltpu.VMEM((PER_W,), jnp.int32),
        pltpu.VMEM((PER_W,), jnp.int32),
    ]
    scratch += [pltpu.VMEM((WIN, PAIR_LANES), jnp.float32) for _ in range(NBUF)]
    scratch += [pltpu.VMEM((WIN * EMBED,), jnp.float32) for _ in range(NBUF)]
    scratch += [pltpu.SemaphoreType.DMA for _ in range(NBUF)]

    @pl.kernel(
        out_type=jax.ShapeDtypeStruct((NUM_IDX * EMBED,), jnp.float32),
        mesh=mesh,
        scratch_types=scratch,
    )
    def k(table_hbm, ih_hbm, ho_hbm, out_hbm, ih_v, ho_v, *rest):
        wbuf = rest[:NBUF]
        obuf = rest[NBUF:2 * NBUF]
        gsem = rest[2 * NBUF:3 * NBUF]

        wid = lax.axis_index("s") * NC + lax.axis_index("c")
        base = wid * PER_W
        pltpu.sync_copy(ih_hbm.at[pl.ds(base, PER_W)], ih_v)
        pltpu.sync_copy(ho_hbm.at[pl.ds(base, PER_W)], ho_v)

        def compact(j, w):
            @pl.loop(0, WIN, step=VREG)
            def _(r0):
                offs = ho_v[pl.ds(w * WIN + r0, VREG)]
                for l in range(VREG):
                    off = offs[l]
                    for c in range(0, EMBED, VREG):
                        obuf[j][pl.ds((r0 + l) * EMBED + c, VREG)] = wbuf[j][
                            r0 + l, pl.ds(off + c, VREG)
                        ]

        @pl.loop(0, NWIN, step=NBUF)
        def _(g):
            handles = []
            for j in range(NBUF):
                h = pltpu.async_copy(
                    table_hbm.at[ih_v.at[pl.ds((g + j) * WIN, WIN)]],
                    wbuf[j],
                    gsem[j],
                )
                handles.append(h)
            for j in range(NBUF):
                handles[j].wait()
                compact(j, g + j)
                pltpu.sync_copy(
                    obuf[j],
                    out_hbm.at[
                        pl.ds((base + (g + j) * WIN) * EMBED, WIN * EMBED)
                    ],
                )

    return k(table_pairs, idx_half, half_off)


def kernel(cat_seq, embed_cat):
    idx = cat_seq.reshape(NUM_IDX).astype(jnp.int32)
    idx_half = idx >> 1
    half_off = (idx & 1) * EMBED
    table_pairs = embed_cat.reshape(VOCAB // 2, PAIR_LANES)
    out = _sc_gather(table_pairs, idx_half, half_off)
    return out.reshape(B, L, EMBED)


# speedup vs baseline: 1.1745x; 1.1745x over previous
import jax
import jax.numpy as jnp
from jax import lax
from jax.experimental import pallas as pl
from jax.experimental.pallas import tpu as pltpu
from jax.experimental.pallas import tpu_sc as plsc

VOCAB = 1000000
EMBED = 64
B = 4096
L = 200

NUM_IDX = B * L
NC, NS = 2, 16
NW = NC * NS
PER_W = NUM_IDX // NW
WIN = 128
NWIN = PER_W // WIN
NBUF = 4
PAIR_LANES = 2 * EMBED
VREG = 16

assert NWIN % NBUF == 0


def _sc_gather(table_pairs, idx):
    mesh = plsc.VectorSubcoreMesh(core_axis_name="c", subcore_axis_name="s")

    scratch = [pltpu.VMEM((PER_W,), jnp.int32)]
    scratch += [pltpu.VMEM((WIN,), jnp.int32) for _ in range(NBUF)]
    scratch += [pltpu.VMEM((WIN, PAIR_LANES), jnp.float32) for _ in range(NBUF)]
    scratch += [
        pltpu.VMEM((WIN // 2, PAIR_LANES), jnp.float32) for _ in range(NBUF)
    ]
    scratch += [pltpu.SemaphoreType.DMA for _ in range(2 * NBUF)]

    @pl.kernel(
        out_type=jax.ShapeDtypeStruct((NUM_IDX // 2, PAIR_LANES), jnp.float32),
        mesh=mesh,
        scratch_types=scratch,
    )
    def k(table_hbm, idx_hbm, out_hbm, idx_v, *rest):
        idx2 = rest[:NBUF]
        wbuf = rest[NBUF:2 * NBUF]
        obuf = rest[2 * NBUF:3 * NBUF]
        gsem = rest[3 * NBUF:4 * NBUF]
        ssem = rest[4 * NBUF:5 * NBUF]

        wid = lax.axis_index("s") * NC + lax.axis_index("c")
        base = wid * PER_W
        base2 = wid * (PER_W // 2)
        pltpu.sync_copy(idx_hbm.at[pl.ds(base, PER_W)], idx_v)

        def fill(j, w):
            @pl.loop(0, WIN, step=VREG)
            def _(c):
                idx2[j][pl.ds(c, VREG)] = idx_v[pl.ds(w * WIN + c, VREG)] >> 1

        def gfire(j):
            pltpu.async_copy(table_hbm.at[idx2[j]], wbuf[j], gsem[j])

        def gwait(j):
            pltpu.make_async_copy(
                table_hbm.at[idx2[j]], wbuf[j], gsem[j]
            ).wait()

        def out_slice(w):
            return out_hbm.at[pl.ds(base2 + w * (WIN // 2), WIN // 2)]

        def sfire(j, w):
            pltpu.async_copy(obuf[j], out_slice(w), ssem[j])

        def swait(j, w):
            pltpu.make_async_copy(obuf[j], out_slice(w), ssem[j]).wait()

        def compact(j, w):
            @pl.loop(0, WIN // 2, step=VREG // 2)
            def _(o0):
                r0 = o0 * 2
                vals = idx_v[pl.ds(w * WIN + r0, VREG)]
                for l in range(VREG):
                    off = (vals[l] & 1) * EMBED
                    for c in range(0, EMBED, VREG):
                        obuf[j][
                            o0 + l // 2, pl.ds((l % 2) * EMBED + c, VREG)
                        ] = wbuf[j][r0 + l, pl.ds(off + c, VREG)]

        for j in range(NBUF):
            fill(j, j)
            gfire(j)
        for j in range(NBUF):
            gwait(j)
            compact(j, j)
            sfire(j, j)
            fill(j, NBUF + j)
            gfire(j)

        @pl.loop(NBUF, NWIN - NBUF, step=NBUF)
        def _(g):
            for j in range(NBUF):
                w = g + j
                swait(j, w - NBUF)
                gwait(j)
                compact(j, w)
                sfire(j, w)
                fill(j, w + NBUF)
                gfire(j)

        for j in range(NBUF):
            w = NWIN - NBUF + j
            swait(j, w - NBUF)
            gwait(j)
            compact(j, w)
            sfire(j, w)
        for j in range(NBUF):
            swait(j, NWIN - NBUF + j)

    return k(table_pairs, idx)


def kernel(cat_seq, embed_cat):
    idx = cat_seq.reshape(NUM_IDX).astype(jnp.int32)
    table_pairs = embed_cat.reshape(VOCAB // 2, PAIR_LANES)
    out = _sc_gather(table_pairs, idx)
    return out.reshape(B, L, EMBED)


# speedup vs baseline: 1.7165x; 1.4615x over previous
import jax
import jax.numpy as jnp
from jax import lax
from jax.experimental import pallas as pl
from jax.experimental.pallas import tpu as pltpu
from jax.experimental.pallas import tpu_sc as plsc

VOCAB = 1000000
EMBED = 64
B = 4096
L = 200

NUM_IDX = B * L
NC, NS = 2, 16
NW = NC * NS
PER_W = NUM_IDX // NW
WIN = 128
NWIN = PER_W // WIN
NBUF = 2
LANES = 2 * EMBED
VREG = 16

assert NWIN % NBUF == 0


def _sc_gather(table128, idx):
    mesh = plsc.VectorSubcoreMesh(core_axis_name="c", subcore_axis_name="s")

    scratch = [pltpu.VMEM((PER_W,), jnp.int32)]
    scratch += [pltpu.VMEM((WIN, LANES), jnp.float32) for _ in range(NBUF)]
    scratch += [pltpu.VMEM((WIN, EMBED), jnp.float32) for _ in range(NBUF)]
    scratch += [pltpu.SemaphoreType.DMA for _ in range(2 * NBUF)]

    @pl.kernel(
        out_type=jax.ShapeDtypeStruct((NUM_IDX, EMBED), jnp.float32),
        mesh=mesh,
        scratch_types=scratch,
    )
    def k(table_hbm, idx_hbm, out_hbm, idx_v, *rest):
        wbuf = rest[:NBUF]
        obuf = rest[NBUF:2 * NBUF]
        gsem = rest[2 * NBUF:3 * NBUF]
        ssem = rest[3 * NBUF:4 * NBUF]

        wid = lax.axis_index("s") * NC + lax.axis_index("c")
        base = wid * PER_W
        pltpu.sync_copy(idx_hbm.at[pl.ds(base, PER_W)], idx_v)

        def gfire(j, w):
            pltpu.async_copy(
                table_hbm.at[idx_v.at[pl.ds(w * WIN, WIN)]], wbuf[j], gsem[j]
            )

        def gwait(j, w):
            pltpu.make_async_copy(
                table_hbm.at[idx_v.at[pl.ds(w * WIN, WIN)]], wbuf[j], gsem[j]
            ).wait()

        def sfire(j, w):
            pltpu.async_copy(
                obuf[j], out_hbm.at[pl.ds(base + w * WIN, WIN)], ssem[j]
            )

        def swait(j, w):
            pltpu.make_async_copy(
                obuf[j], out_hbm.at[pl.ds(base + w * WIN, WIN)], ssem[j]
            ).wait()

        def compact(j):
            @pl.loop(0, WIN)
            def _(r):
                for c in range(0, EMBED, VREG):
                    obuf[j][r, pl.ds(c, VREG)] = wbuf[j][r, pl.ds(c, VREG)]

        for j in range(NBUF):
            gfire(j, j)
        for j in range(NBUF):
            gwait(j, j)
            compact(j)
            sfire(j, j)
            gfire(j, NBUF + j)

        @pl.loop(NBUF, NWIN - NBUF, step=NBUF)
        def _(g):
            for j in range(NBUF):
                w = g + j
                swait(j, w - NBUF)
                gwait(j, w)
                compact(j)
                sfire(j, w)
                gfire(j, w + NBUF)

        for j in range(NBUF):
            w = NWIN - NBUF + j
            swait(j, w - NBUF)
            gwait(j, w)
            compact(j)
            sfire(j, w)
        for j in range(NBUF):
            swait(j, NWIN - NBUF + j)

    return k(table128, idx)


def kernel(cat_seq, embed_cat):
    idx = cat_seq.reshape(NUM_IDX).astype(jnp.int32)
    table128 = jnp.pad(embed_cat, ((0, 0), (0, LANES - EMBED)))
    out = _sc_gather(table128, idx)
    return out.reshape(B, L, EMBED)
